# baseline (speedup 1.0000x reference)
import functools

import jax
import jax.numpy as jnp
from jax.experimental import pallas as pl
from jax.experimental.pallas import tpu as pltpu

_IN_F = 8
_HID = 12


def _mlp_mxu2(xt_ref, w1_ref, b1_ref, w2_ref, b2_ref, o_ref):
    x = xt_ref[...]
    h = jax.lax.dot_general(
        w1_ref[...], x,
        dimension_numbers=(((1,), (0,)), ((), ())),
        preferred_element_type=jnp.float32,
        precision=jax.lax.Precision.DEFAULT,
    )
    h = jnp.maximum(h + b1_ref[...], 0.0)
    z8 = jax.lax.dot_general(
        w2_ref[...], h,
        dimension_numbers=(((1,), (0,)), ((), ())),
        preferred_element_type=jnp.float32,
        precision=jax.lax.Precision.DEFAULT,
    )
    o_ref[...] = jax.nn.sigmoid(z8[0, :] + b2_ref[0])


@functools.partial(jax.jit, static_argnames=("lane_block",))
def _forward(x, w1, b1, w2, b2, *, lane_block=262144):
    B = x.shape[0]
    xt = x.astype(jnp.float32).T
    pad = -B % 128
    if pad:
        xt = jnp.pad(xt, ((0, 0), (0, pad)))
    n = xt.shape[1]

    w1p = jnp.zeros((16, _IN_F), jnp.float32).at[:_HID].set(
        w1.astype(jnp.float32))
    b1p = jnp.zeros((16, 1), jnp.float32).at[:_HID, 0].set(
        b1.astype(jnp.float32))
    w2p = jnp.tile(
        jnp.zeros((1, 16), jnp.float32).at[0, :_HID].set(
            w2.astype(jnp.float32).reshape(_HID)),
        (8, 1))

    lb = min(lane_block, n)
    grid = (pl.cdiv(n, lb),)

    out_flat = pl.pallas_call(
        _mlp_mxu2,
        out_shape=jax.ShapeDtypeStruct((n,), jnp.float32),
        grid=grid,
        in_specs=[
            pl.BlockSpec((_IN_F, lb), lambda i: (0, i)),
            pl.BlockSpec((16, _IN_F), lambda i: (0, 0)),
            pl.BlockSpec((16, 1), lambda i: (0, 0)),
            pl.BlockSpec((8, 16), lambda i: (0, 0)),
            pl.BlockSpec(memory_space=pltpu.MemorySpace.SMEM),
        ],
        out_specs=pl.BlockSpec((lb,), lambda i: (i,)),
        compiler_params=pltpu.CompilerParams(
            dimension_semantics=("parallel",),
        ),
    )(
        xt,
        w1p,
        b1p,
        w2p,
        b2.astype(jnp.float32),
    )

    return out_flat[:B].reshape(B, 1)


def kernel(x, w1, b1, w2, b2):
    return _forward(x, w1, b1, w2, b2)


# speedup vs baseline: 1.1126x; 1.1126x over previous
import functools

import jax
import jax.numpy as jnp
from jax.experimental import pallas as pl
from jax.experimental.pallas import tpu as pltpu

_IN_F = 8
_HID = 12


def _mlp_mxu2(xt_ref, p_ref, b2_ref, o_ref):
    x = xt_ref[...]
    pv = p_ref[...]
    w1v = pv[:, :_IN_F]
    b1v = pv[:, _IN_F:_IN_F + 1]
    w2v = pv[:8, 16:]
    h = jax.lax.dot_general(
        w1v, x,
        dimension_numbers=(((1,), (0,)), ((), ())),
        preferred_element_type=jnp.float32,
        precision=jax.lax.Precision.DEFAULT,
    )
    h = jnp.maximum(h + b1v, 0.0)
    z8 = jax.lax.dot_general(
        w2v, h,
        dimension_numbers=(((1,), (0,)), ((), ())),
        preferred_element_type=jnp.float32,
        precision=jax.lax.Precision.DEFAULT,
    )
    o_ref[...] = jax.nn.sigmoid(z8[0, :] + b2_ref[0])


@functools.partial(jax.jit, static_argnames=("lane_block",))
def _forward(x, w1, b1, w2, b2, *, lane_block=131072):
    B = x.shape[0]
    xt = x.astype(jnp.float32).T
    pad = -B % 128
    if pad:
        xt = jnp.pad(xt, ((0, 0), (0, pad)))
    n = xt.shape[1]

    w1f = w1.astype(jnp.float32)
    a = jnp.pad(
        jnp.concatenate([w1f, b1.astype(jnp.float32)[:, None]], axis=1),
        ((0, 4), (0, 7)))
    w2rep = jnp.pad(
        jnp.broadcast_to(
            jnp.pad(w2.astype(jnp.float32).reshape(1, _HID), ((0, 0), (0, 4))),
            (8, 16)),
        ((0, 8), (0, 0)))
    packed = jnp.concatenate([a, w2rep], axis=1)

    lb = min(lane_block, n)
    grid = (pl.cdiv(n, lb),)

    out_flat = pl.pallas_call(
        _mlp_mxu2,
        out_shape=jax.ShapeDtypeStruct((n,), jnp.float32),
        grid=grid,
        in_specs=[
            pl.BlockSpec((_IN_F, lb), lambda i: (0, i)),
            pl.BlockSpec((16, 32), lambda i: (0, 0)),
            pl.BlockSpec(memory_space=pltpu.MemorySpace.SMEM),
        ],
        out_specs=pl.BlockSpec((lb,), lambda i: (i,)),
        compiler_params=pltpu.CompilerParams(
            dimension_semantics=("parallel",),
        ),
    )(xt, packed, b2.astype(jnp.float32))

    return out_flat[:B].reshape(B, 1)


def kernel(x, w1, b1, w2, b2):
    return _forward(x, w1, b1, w2, b2)


# speedup vs baseline: 1.2493x; 1.1229x over previous
import functools

import jax
import jax.numpy as jnp
from jax.experimental import pallas as pl
from jax.experimental.pallas import tpu as pltpu

_IN_F = 8
_HID = 12


def _mlp_mxu2(xt_ref, p_ref, b2_ref, o_ref, zd_ref):
    x = xt_ref[...]
    pv = p_ref[...]
    w1v = pv[:, :_IN_F]
    b1v = pv[:, _IN_F:_IN_F + 1]
    w2v = pv[:8, 16:]
    h = jax.lax.dot_general(
        w1v, x,
        dimension_numbers=(((1,), (0,)), ((), ())),
        preferred_element_type=jnp.float32,
        precision=jax.lax.Precision.DEFAULT,
    )
    h = jnp.maximum(h + b1v, 0.0)
    z8 = jax.lax.dot_general(
        w2v, h,
        dimension_numbers=(((1,), (0,)), ((), ())),
        preferred_element_type=jnp.float32,
        precision=jax.lax.Precision.DEFAULT,
    )
    zd_ref[...] = z8[0, :]
    o_ref[...] = jax.nn.sigmoid(zd_ref[...] + b2_ref[0])


@functools.partial(jax.jit, static_argnames=("lane_block",))
def _forward(x, w1, b1, w2, b2, *, lane_block=131072):
    B = x.shape[0]
    xt = x.astype(jnp.float32).T
    pad = -B % 128
    if pad:
        xt = jnp.pad(xt, ((0, 0), (0, pad)))
    n = xt.shape[1]

    w1f = w1.astype(jnp.float32)
    a = jnp.pad(
        jnp.concatenate([w1f, b1.astype(jnp.float32)[:, None]], axis=1),
        ((0, 4), (0, 7)))
    w2rep = jnp.pad(
        jnp.broadcast_to(
            jnp.pad(w2.astype(jnp.float32).reshape(1, _HID), ((0, 0), (0, 4))),
            (8, 16)),
        ((0, 8), (0, 0)))
    packed = jnp.concatenate([a, w2rep], axis=1)

    lb = min(lane_block, n)
    grid = (pl.cdiv(n, lb),)

    out_flat = pl.pallas_call(
        _mlp_mxu2,
        out_shape=jax.ShapeDtypeStruct((n,), jnp.float32),
        grid=grid,
        in_specs=[
            pl.BlockSpec((_IN_F, lb), lambda i: (0, i)),
            pl.BlockSpec((16, 32), lambda i: (0, 0)),
            pl.BlockSpec(memory_space=pltpu.MemorySpace.SMEM),
        ],
        out_specs=pl.BlockSpec((lb,), lambda i: (i,)),
        scratch_shapes=[pltpu.VMEM((lb,), jnp.float32)],
        compiler_params=pltpu.CompilerParams(
            dimension_semantics=("parallel",),
        ),
    )(xt, packed, b2.astype(jnp.float32))

    return out_flat[:B].reshape(B, 1)


def kernel(x, w1, b1, w2, b2):
    return _forward(x, w1, b1, w2, b2)


# speedup vs baseline: 1.3942x; 1.1160x over previous
import functools

import jax
import jax.numpy as jnp
from jax.experimental import pallas as pl
from jax.experimental.pallas import tpu as pltpu

_IN_F = 8
_HID = 12


def _mlp_mxu2(xt_ref, w1t_ref, b1_ref, w2_ref, b2_ref, o_ref, zd_ref):
    x = xt_ref[...]
    h = jax.lax.dot_general(
        w1t_ref[...], x,
        dimension_numbers=(((0,), (0,)), ((), ())),
        preferred_element_type=jnp.float32,
        precision=jax.lax.Precision.DEFAULT,
    )
    h = jnp.maximum(h + b1_ref[...].T, 0.0)
    z8 = jax.lax.dot_general(
        jnp.broadcast_to(w2_ref[...], (8, _HID)), h,
        dimension_numbers=(((1,), (0,)), ((), ())),
        preferred_element_type=jnp.float32,
        precision=jax.lax.Precision.DEFAULT,
    )
    zd_ref[...] = z8[0, :]
    o_ref[...] = jax.nn.sigmoid(zd_ref[...] + b2_ref[0])


@functools.partial(jax.jit, static_argnames=("lane_block",))
def _forward(x, w1, b1, w2, b2, *, lane_block=131072):
    B = x.shape[0]
    xt = x.astype(jnp.float32).T
    pad = -B % 128
    if pad:
        xt = jnp.pad(xt, ((0, 0), (0, pad)))
    n = xt.shape[1]

    lb = min(lane_block, n)
    grid = (pl.cdiv(n, lb),)

    out_flat = pl.pallas_call(
        _mlp_mxu2,
        out_shape=jax.ShapeDtypeStruct((n,), jnp.float32),
        grid=grid,
        in_specs=[
            pl.BlockSpec((_IN_F, lb), lambda i: (0, i)),
            pl.BlockSpec((_IN_F, _HID), lambda i: (0, 0)),
            pl.BlockSpec((1, _HID), lambda i: (0, 0)),
            pl.BlockSpec((1, _HID), lambda i: (0, 0)),
            pl.BlockSpec(memory_space=pltpu.MemorySpace.SMEM),
        ],
        out_specs=pl.BlockSpec((lb,), lambda i: (i,)),
        scratch_shapes=[pltpu.VMEM((lb,), jnp.float32)],
        compiler_params=pltpu.CompilerParams(
            dimension_semantics=("parallel",),
        ),
    )(
        xt,
        w1.astype(jnp.float32).T,
        b1.astype(jnp.float32).reshape(1, _HID),
        w2.astype(jnp.float32),
        b2.astype(jnp.float32),
    )

    return out_flat[:B].reshape(B, 1)


def kernel(x, w1, b1, w2, b2):
    return _forward(x, w1, b1, w2, b2)


# speedup vs baseline: 1.4627x; 1.0492x over previous
import functools

import jax
import jax.numpy as jnp
from jax.experimental import pallas as pl
from jax.experimental.pallas import tpu as pltpu

_IN_F = 8
_HID = 12


def _mlp_mxu2(xt_ref, w1t_ref, b1_ref, w2_ref, b2_ref, o_ref, zd_ref):
    x = xt_ref[...]
    h = jax.lax.dot_general(
        w1t_ref[...], x,
        dimension_numbers=(((0,), (0,)), ((), ())),
        preferred_element_type=jnp.float32,
        precision=jax.lax.Precision.DEFAULT,
    )
    h = jnp.maximum(h + b1_ref[...].T, 0.0)
    z8 = jax.lax.dot_general(
        jnp.broadcast_to(w2_ref[...], (8, _HID)), h,
        dimension_numbers=(((1,), (0,)), ((), ())),
        preferred_element_type=jnp.float32,
        precision=jax.lax.Precision.DEFAULT,
    )
    zd_ref[...] = z8[0, :]
    o_ref[...] = jax.nn.sigmoid(zd_ref[...] + b2_ref[0])


@functools.partial(jax.jit, static_argnames=("lane_block",))
def _forward(x, w1, b1, w2, b2, *, lane_block=262144):
    B = x.shape[0]
    xt = x.astype(jnp.float32).T
    pad = -B % 128
    if pad:
        xt = jnp.pad(xt, ((0, 0), (0, pad)))
    n = xt.shape[1]

    lb = min(lane_block, n)
    grid = (pl.cdiv(n, lb),)

    out_flat = pl.pallas_call(
        _mlp_mxu2,
        out_shape=jax.ShapeDtypeStruct((n,), jnp.float32),
        grid=grid,
        in_specs=[
            pl.BlockSpec((_IN_F, lb), lambda i: (0, i)),
            pl.BlockSpec((_IN_F, _HID), lambda i: (0, 0)),
            pl.BlockSpec((1, _HID), lambda i: (0, 0)),
            pl.BlockSpec((1, _HID), lambda i: (0, 0)),
            pl.BlockSpec(memory_space=pltpu.MemorySpace.SMEM),
        ],
        out_specs=pl.BlockSpec((lb,), lambda i: (i,)),
        scratch_shapes=[pltpu.VMEM((lb,), jnp.float32)],
        compiler_params=pltpu.CompilerParams(
            dimension_semantics=("parallel",),
        ),
    )(
        xt,
        w1.astype(jnp.float32).T,
        b1.astype(jnp.float32).reshape(1, _HID),
        w2.astype(jnp.float32),
        b2.astype(jnp.float32),
    )

    return out_flat[:B].reshape(B, 1)


def kernel(x, w1, b1, w2, b2):
    return _forward(x, w1, b1, w2, b2)
